# Initial kernel scaffold; baseline (speedup 1.0000x reference)
import jax
import jax.numpy as jnp
from jax.experimental import pallas as pl


def kernel(queries, keys, values):
    raise NotImplementedError("write your pallas kernel here")


# speedup vs baseline: 1.4650x; 1.4650x over previous
import functools

import numpy as np
import jax
import jax.numpy as jnp
from jax import lax
from jax.experimental import pallas as pl
from jax.experimental.pallas import tpu as pltpu
from jax.experimental.pallas import tpu_sc as plsc

B = 4
L = 4096
D = 1024
KTOP = 5
FP = 2112
WBLK = 176
NWB = FP // WBLK

_PREC = lax.Precision.HIGHEST


@functools.lru_cache(maxsize=None)
def _dft_consts():
    f = np.arange(FP, dtype=np.float64)[:, None]
    t = np.arange(L, dtype=np.float64)[None, :]
    ang = 2.0 * np.pi * f * t / L
    wc = np.cos(ang)
    ws = np.sin(ang)
    wc[2049:] = 0.0
    ws[2049:] = 0.0
    cv = np.full(FP, 2.0)
    cv[0] = 1.0
    cv[2048] = 1.0
    cv[2049:] = 0.0
    cv /= float(L) * float(D)
    cv3 = cv.reshape(NWB, 1, WBLK)
    return (wc.astype(np.float32), ws.astype(np.float32),
            cv3.astype(np.float32))


def _dot(a, b):
    return lax.dot_general(a, b, (((1,), (0,)), ((), ())),
                           precision=_PREC,
                           preferred_element_type=jnp.float32)


DBLK = 256
NDB = D // DBLK


def _spectrum_body(q_ref, k_ref, wc_ref, ws_ref, sr_ref, si_ref,
                   asr_ref, asi_ref):
    d2 = pl.program_id(1)
    w = pl.program_id(2)
    q = q_ref[0]
    k = k_ref[0]
    wc = wc_ref[...]
    ws = ws_ref[...]
    qr = _dot(wc, q)
    qs = _dot(ws, q)
    kr = _dot(wc, k)
    ks = _dot(ws, k)
    sr = jnp.sum(qr * kr + qs * ks, axis=1, keepdims=True)
    si = jnp.sum(qr * ks - qs * kr, axis=1, keepdims=True)
    rows = pl.ds(w * WBLK, WBLK)

    @pl.when(d2 == 0)
    def _():
        asr_ref[rows, :] = sr
        asi_ref[rows, :] = si

    @pl.when(d2 > 0)
    def _():
        asr_ref[rows, :] = asr_ref[rows, :] + sr
        asi_ref[rows, :] = asi_ref[rows, :] + si

    @pl.when(d2 == NDB - 1)
    def _():
        sr_ref[0] = asr_ref[rows, :]
        si_ref[0] = asi_ref[rows, :]


def _cross_spectrum(q, k, wc, ws):
    grid = (B, NDB, NWB)
    return pl.pallas_call(
        _spectrum_body,
        grid=grid,
        in_specs=[
            pl.BlockSpec((1, L, DBLK), lambda b, d2, w: (b, 0, d2)),
            pl.BlockSpec((1, L, DBLK), lambda b, d2, w: (b, 0, d2)),
            pl.BlockSpec((WBLK, L), lambda b, d2, w: (w, 0)),
            pl.BlockSpec((WBLK, L), lambda b, d2, w: (w, 0)),
        ],
        out_specs=[
            pl.BlockSpec((1, WBLK, 1), lambda b, d2, w: (b, w, 0)),
            pl.BlockSpec((1, WBLK, 1), lambda b, d2, w: (b, w, 0)),
        ],
        out_shape=[
            jax.ShapeDtypeStruct((B, FP, 1), jnp.float32),
            jax.ShapeDtypeStruct((B, FP, 1), jnp.float32),
        ],
        scratch_shapes=[
            pltpu.VMEM((FP, 1), jnp.float32),
            pltpu.VMEM((FP, 1), jnp.float32),
        ],
    )(q, k, wc, ws)


def _select_body(sr_ref, si_ref, cv_ref, wc_ref, ws_ref,
                 ti_ref, wt_ref, acc_ref):
    w = pl.program_id(0)
    srb = sr_ref[0] * cv_ref[0]
    sib = si_ref[0] * cv_ref[0]
    contrib = _dot(srb, wc_ref[...]) - _dot(sib, ws_ref[...])

    @pl.when(w == 0)
    def _():
        acc_ref[...] = contrib

    @pl.when(w > 0)
    def _():
        acc_ref[...] = acc_ref[...] + contrib

    @pl.when(w == NWB - 1)
    def _():
        scores = acc_ref[...]
        iota = lax.broadcasted_iota(jnp.int32, (B, L), 1)
        cur = scores
        vals, idxs = [], []
        for _ in range(KTOP):
            m = jnp.max(cur, axis=1, keepdims=True)
            hit = cur == m
            idx = jnp.min(jnp.where(hit, iota, L), axis=1, keepdims=True)
            vals.append(m)
            idxs.append(idx)
            cur = jnp.where(iota == idx, -1e30, cur)
        pad_v = [jnp.full((B, 1), -1e30, jnp.float32)] * (8 - KTOP)
        pad_i = [jnp.zeros((B, 1), jnp.int32)] * (8 - KTOP)
        v = jnp.concatenate(vals + pad_v, axis=1)
        ix = jnp.concatenate(idxs + pad_i, axis=1)
        mx = jnp.max(v, axis=1, keepdims=True)
        e = jnp.exp(v - mx)
        wt = e / jnp.sum(e, axis=1, keepdims=True)
        s_b = lax.broadcast_in_dim(ix, (B, 8, L), (0, 1))
        l_i = lax.broadcasted_iota(jnp.int32, (B, 8, L), 2)
        b_i = lax.broadcasted_iota(jnp.int32, (B, 8, L), 0)
        ti_ref[...] = b_i * L + lax.rem(l_i - s_b + L, L)
        wt_ref[...] = lax.broadcast_in_dim(wt, (B, 8, 16), (0, 1))


def _select(sr, si, cv, wc, ws):
    grid = (NWB,)
    return pl.pallas_call(
        _select_body,
        grid=grid,
        in_specs=[
            pl.BlockSpec((1, B, WBLK), lambda w: (w, 0, 0)),
            pl.BlockSpec((1, B, WBLK), lambda w: (w, 0, 0)),
            pl.BlockSpec((1, 1, WBLK), lambda w: (w, 0, 0)),
            pl.BlockSpec((WBLK, L), lambda w: (w, 0)),
            pl.BlockSpec((WBLK, L), lambda w: (w, 0)),
        ],
        out_specs=[
            pl.BlockSpec((B, 8, L), lambda w: (0, 0, 0)),
            pl.BlockSpec((B, 8, 16), lambda w: (0, 0, 0)),
        ],
        out_shape=[
            jax.ShapeDtypeStruct((B, 8, L), jnp.int32),
            jax.ShapeDtypeStruct((B, 8, 16), jnp.float32),
        ],
        scratch_shapes=[pltpu.VMEM((B, L), jnp.float32)],
    )(sr, si, cv, wc, ws)


NWORK = 32
RPW = (B * L) // NWORK
CH = 32
NCH = RPW // CH


def _roll_body(vals_hbm, im_hbm, w_hbm, out_hbm,
               w0_v, w1_v, w2_v, w3_v, w4_v, idx_v, buf, acc, sem):
    cid = lax.axis_index("c")
    sid = lax.axis_index("s")
    wid = sid * 2 + cid
    b = wid // (NWORK // B)
    blk = wid % (NWORK // B)
    w_slots = [w0_v, w1_v, w2_v, w3_v, w4_v]
    for i in range(KTOP):
        pltpu.sync_copy(w_hbm.at[b * 8 + i], w_slots[i])
    wvs = [w_slots[i][...] for i in range(KTOP)]

    def chunk_body(ci, carry):
        base_l = blk * RPW + ci * CH
        for i in range(KTOP):
            pltpu.sync_copy(
                im_hbm.at[pl.ds((b * 8 + i) * L + blk * RPW + ci * CH, CH)],
                idx_v)
            pltpu.async_copy(vals_hbm.at[idx_v], buf, sem).wait()
            wv = wvs[i]

            def row_body(r, c2):
                for g in range(D // 16):
                    x = buf[r, pl.ds(g * 16, 16)] * wv
                    if i == 0:
                        acc[r, pl.ds(g * 16, 16)] = x
                    else:
                        acc[r, pl.ds(g * 16, 16)] = (
                            acc[r, pl.ds(g * 16, 16)] + x)
                return c2

            lax.fori_loop(0, CH, row_body, 0)
        pltpu.sync_copy(acc, out_hbm.at[pl.ds(b * L + base_l, CH)])
        return carry

    lax.fori_loop(0, NCH, chunk_body, 0)


def _sc_roll(values2d, idxmap, wts):
    mesh = plsc.VectorSubcoreMesh(core_axis_name="c", subcore_axis_name="s")
    fn = functools.partial(
        pl.kernel,
        mesh=mesh,
        out_type=jax.ShapeDtypeStruct((B * L, D), jnp.float32),
        scratch_types=[
            pltpu.VMEM((16,), jnp.float32),
            pltpu.VMEM((16,), jnp.float32),
            pltpu.VMEM((16,), jnp.float32),
            pltpu.VMEM((16,), jnp.float32),
            pltpu.VMEM((16,), jnp.float32),
            pltpu.VMEM((CH,), jnp.int32),
            pltpu.VMEM((CH, D), jnp.float32),
            pltpu.VMEM((CH, D), jnp.float32),
            pltpu.SemaphoreType.DMA,
        ],
    )(_roll_body)
    return fn(values2d, idxmap, wts)


def kernel(queries, keys, values):
    wc_np, ws_np, cv_np = _dft_consts()
    wc = jnp.asarray(wc_np)
    ws = jnp.asarray(ws_np)
    cv = jnp.asarray(cv_np)
    sr, si = _cross_spectrum(queries, keys, wc, ws)
    sr3 = jnp.transpose(sr.reshape(B, NWB, WBLK), (1, 0, 2))
    si3 = jnp.transpose(si.reshape(B, NWB, WBLK), (1, 0, 2))
    idxmap, wts = _select(sr3, si3, cv, wc, ws)
    out2d = _sc_roll(values.reshape(B * L, D),
                     idxmap.reshape(B * 8 * L), wts.reshape(B * 8, 16))
    return out2d.reshape(B, L, D)


# speedup vs baseline: 2.3373x; 1.5954x over previous
import functools

import numpy as np
import jax
import jax.numpy as jnp
from jax import lax
from jax.experimental import pallas as pl
from jax.experimental.pallas import tpu as pltpu
from jax.experimental.pallas import tpu_sc as plsc

B = 4
L = 4096
D = 1024
KTOP = 5
FP = 2112
WBLK = 176
NWB = FP // WBLK

def _split_np(x):
    hi = x.astype(np.float32).astype(jnp.bfloat16)
    lo = (x.astype(np.float32) - hi.astype(np.float32)).astype(jnp.bfloat16)
    return hi, lo


@functools.lru_cache(maxsize=None)
def _dft_consts():
    f = np.arange(FP, dtype=np.float64)[:, None]
    t = np.arange(L, dtype=np.float64)[None, :]
    ang = 2.0 * np.pi * f * t / L
    wc = np.cos(ang)
    ws = np.sin(ang)
    wc[2049:] = 0.0
    ws[2049:] = 0.0
    cv = np.full(FP, 2.0)
    cv[0] = 1.0
    cv[2048] = 1.0
    cv[2049:] = 0.0
    cv /= float(L) * float(D)
    cv3 = cv.reshape(NWB, 1, WBLK).astype(np.float32)
    wch, wcl = _split_np(wc)
    wsh, wsl = _split_np(ws)
    return wch, wcl, wsh, wsl, cv3


def _dot(a, b):
    return lax.dot_general(a, b, (((1,), (0,)), ((), ())),
                           preferred_element_type=jnp.float32)


def _dot3(ah, al, bh, bl):
    return _dot(ah, bh) + _dot(ah, bl) + _dot(al, bh)


def _split_jx(x):
    hi = x.astype(jnp.bfloat16)
    lo = (x - hi.astype(jnp.float32)).astype(jnp.bfloat16)
    return hi, lo


DBLK = 512
NDB = D // DBLK


def _spectrum_body(qh_ref, ql_ref, kh_ref, kl_ref,
                   wch_ref, wcl_ref, wsh_ref, wsl_ref, sr_ref, si_ref,
                   asr_ref, asi_ref):
    d2 = pl.program_id(1)
    w = pl.program_id(2)
    qh, ql = qh_ref[0], ql_ref[0]
    kh, kl = kh_ref[0], kl_ref[0]
    wch, wcl = wch_ref[...], wcl_ref[...]
    wsh, wsl = wsh_ref[...], wsl_ref[...]
    qr = _dot3(wch, wcl, qh, ql)
    qs = _dot3(wsh, wsl, qh, ql)
    kr = _dot3(wch, wcl, kh, kl)
    ks = _dot3(wsh, wsl, kh, kl)
    sr = jnp.sum(qr * kr + qs * ks, axis=1, keepdims=True)
    si = jnp.sum(qr * ks - qs * kr, axis=1, keepdims=True)
    rows = pl.ds(w * WBLK, WBLK)

    @pl.when(d2 == 0)
    def _():
        asr_ref[rows, :] = sr
        asi_ref[rows, :] = si

    @pl.when(d2 > 0)
    def _():
        asr_ref[rows, :] = asr_ref[rows, :] + sr
        asi_ref[rows, :] = asi_ref[rows, :] + si

    @pl.when(d2 == NDB - 1)
    def _():
        sr_ref[0] = asr_ref[rows, :]
        si_ref[0] = asi_ref[rows, :]


def _cross_spectrum(qh, ql, kh, kl, wch, wcl, wsh, wsl):
    grid = (B, NDB, NWB)
    dspec = pl.BlockSpec((1, L, DBLK), lambda b, d2, w: (b, 0, d2))
    wspec = pl.BlockSpec((WBLK, L), lambda b, d2, w: (w, 0))
    return pl.pallas_call(
        _spectrum_body,
        grid=grid,
        in_specs=[dspec, dspec, dspec, dspec, wspec, wspec, wspec, wspec],
        out_specs=[
            pl.BlockSpec((1, WBLK, 1), lambda b, d2, w: (b, w, 0)),
            pl.BlockSpec((1, WBLK, 1), lambda b, d2, w: (b, w, 0)),
        ],
        out_shape=[
            jax.ShapeDtypeStruct((B, FP, 1), jnp.float32),
            jax.ShapeDtypeStruct((B, FP, 1), jnp.float32),
        ],
        scratch_shapes=[
            pltpu.VMEM((FP, 1), jnp.float32),
            pltpu.VMEM((FP, 1), jnp.float32),
        ],
    )(qh, ql, kh, kl, wch, wcl, wsh, wsl)


def _select_body(sr_ref, si_ref, cv_ref, wch_ref, wcl_ref, wsh_ref, wsl_ref,
                 ti_ref, wt_ref, acc_ref):
    w = pl.program_id(0)
    srh, srl = _split_jx(sr_ref[0] * cv_ref[0])
    sih, sil = _split_jx(si_ref[0] * cv_ref[0])
    contrib = (_dot3(srh, srl, wch_ref[...], wcl_ref[...])
               - _dot3(sih, sil, wsh_ref[...], wsl_ref[...]))

    @pl.when(w == 0)
    def _():
        acc_ref[...] = contrib

    @pl.when(w > 0)
    def _():
        acc_ref[...] = acc_ref[...] + contrib

    @pl.when(w == NWB - 1)
    def _():
        scores = acc_ref[...]
        iota = lax.broadcasted_iota(jnp.int32, (B, L), 1)
        cur = scores
        vals, idxs = [], []
        for _ in range(KTOP):
            m = jnp.max(cur, axis=1, keepdims=True)
            hit = cur == m
            idx = jnp.min(jnp.where(hit, iota, L), axis=1, keepdims=True)
            vals.append(m)
            idxs.append(idx)
            cur = jnp.where(iota == idx, -1e30, cur)
        pad_v = [jnp.full((B, 1), -1e30, jnp.float32)] * (8 - KTOP)
        pad_i = [jnp.zeros((B, 1), jnp.int32)] * (8 - KTOP)
        v = jnp.concatenate(vals + pad_v, axis=1)
        ix = jnp.concatenate(idxs + pad_i, axis=1)
        mx = jnp.max(v, axis=1, keepdims=True)
        e = jnp.exp(v - mx)
        wt = e / jnp.sum(e, axis=1, keepdims=True)
        s_b = lax.broadcast_in_dim(ix, (B, 8, L), (0, 1))
        l_i = lax.broadcasted_iota(jnp.int32, (B, 8, L), 2)
        b_i = lax.broadcasted_iota(jnp.int32, (B, 8, L), 0)
        ti_ref[...] = b_i * L + lax.rem(l_i - s_b + L, L)
        wt_ref[...] = lax.broadcast_in_dim(wt, (B, 8, 16), (0, 1))


def _select(sr, si, cv, wch, wcl, wsh, wsl):
    grid = (NWB,)
    wspec = pl.BlockSpec((WBLK, L), lambda w: (w, 0))
    return pl.pallas_call(
        _select_body,
        grid=grid,
        in_specs=[
            pl.BlockSpec((1, B, WBLK), lambda w: (w, 0, 0)),
            pl.BlockSpec((1, B, WBLK), lambda w: (w, 0, 0)),
            pl.BlockSpec((1, 1, WBLK), lambda w: (w, 0, 0)),
            wspec, wspec, wspec, wspec,
        ],
        out_specs=[
            pl.BlockSpec((B, 8, L), lambda w: (0, 0, 0)),
            pl.BlockSpec((B, 8, 16), lambda w: (0, 0, 0)),
        ],
        out_shape=[
            jax.ShapeDtypeStruct((B, 8, L), jnp.int32),
            jax.ShapeDtypeStruct((B, 8, 16), jnp.float32),
        ],
        scratch_shapes=[pltpu.VMEM((B, L), jnp.float32)],
    )(sr, si, cv, wch, wcl, wsh, wsl)


NWORK = 32
RPW = (B * L) // NWORK
CH = 32
NCH = RPW // CH


def _roll_body(vals_hbm, im_hbm, w_hbm, out_hbm,
               w0_v, w1_v, w2_v, w3_v, w4_v, idx_v, buf, acc, sem):
    cid = lax.axis_index("c")
    sid = lax.axis_index("s")
    wid = sid * 2 + cid
    b = wid // (NWORK // B)
    blk = wid % (NWORK // B)
    w_slots = [w0_v, w1_v, w2_v, w3_v, w4_v]
    for i in range(KTOP):
        pltpu.sync_copy(w_hbm.at[b * 8 + i], w_slots[i])
    wvs = [w_slots[i][...] for i in range(KTOP)]

    def chunk_body(ci, carry):
        base_l = blk * RPW + ci * CH
        for i in range(KTOP):
            pltpu.sync_copy(
                im_hbm.at[pl.ds((b * 8 + i) * L + blk * RPW + ci * CH, CH)],
                idx_v)
            pltpu.async_copy(vals_hbm.at[idx_v], buf, sem).wait()
            wv = wvs[i]

            def row_body(r, c2):
                for g in range(D // 16):
                    x = buf[r, pl.ds(g * 16, 16)] * wv
                    if i == 0:
                        acc[r, pl.ds(g * 16, 16)] = x
                    else:
                        acc[r, pl.ds(g * 16, 16)] = (
                            acc[r, pl.ds(g * 16, 16)] + x)
                return c2

            lax.fori_loop(0, CH, row_body, 0)
        pltpu.sync_copy(acc, out_hbm.at[pl.ds(b * L + base_l, CH)])
        return carry

    lax.fori_loop(0, NCH, chunk_body, 0)


def _sc_roll(values2d, idxmap, wts):
    mesh = plsc.VectorSubcoreMesh(core_axis_name="c", subcore_axis_name="s")
    fn = functools.partial(
        pl.kernel,
        mesh=mesh,
        out_type=jax.ShapeDtypeStruct((B * L, D), jnp.float32),
        scratch_types=[
            pltpu.VMEM((16,), jnp.float32),
            pltpu.VMEM((16,), jnp.float32),
            pltpu.VMEM((16,), jnp.float32),
            pltpu.VMEM((16,), jnp.float32),
            pltpu.VMEM((16,), jnp.float32),
            pltpu.VMEM((CH,), jnp.int32),
            pltpu.VMEM((CH, D), jnp.float32),
            pltpu.VMEM((CH, D), jnp.float32),
            pltpu.SemaphoreType.DMA,
        ],
    )(_roll_body)
    return fn(values2d, idxmap, wts)


def kernel(queries, keys, values):
    wch_np, wcl_np, wsh_np, wsl_np, cv_np = _dft_consts()
    wch, wcl = jnp.asarray(wch_np), jnp.asarray(wcl_np)
    wsh, wsl = jnp.asarray(wsh_np), jnp.asarray(wsl_np)
    cv = jnp.asarray(cv_np)
    qh, ql = _split_jx(queries)
    kh, kl = _split_jx(keys)
    sr, si = _cross_spectrum(qh, ql, kh, kl, wch, wcl, wsh, wsl)
    sr3 = jnp.transpose(sr.reshape(B, NWB, WBLK), (1, 0, 2))
    si3 = jnp.transpose(si.reshape(B, NWB, WBLK), (1, 0, 2))
    idxmap, wts = _select(sr3, si3, cv, wch, wcl, wsh, wsl)
    out2d = _sc_roll(values.reshape(B * L, D),
                     idxmap.reshape(B * 8 * L), wts.reshape(B * 8, 16))
    return out2d.reshape(B, L, D)


# speedup vs baseline: 2.5163x; 1.0766x over previous
import functools

import numpy as np
import jax
import jax.numpy as jnp
from jax import lax
from jax.experimental import pallas as pl
from jax.experimental.pallas import tpu as pltpu
from jax.experimental.pallas import tpu_sc as plsc

B = 4
L = 4096
D = 1024
KTOP = 5
FP = 2112
WBLK = 176
NWB = FP // WBLK

def _split_np(x):
    hi = x.astype(np.float32).astype(jnp.bfloat16)
    lo = (x.astype(np.float32) - hi.astype(np.float32)).astype(jnp.bfloat16)
    return hi, lo


@functools.lru_cache(maxsize=None)
def _dft_consts():
    f = np.arange(FP, dtype=np.float64)[:, None]
    t = np.arange(L, dtype=np.float64)[None, :]
    ang = 2.0 * np.pi * f * t / L
    wc = np.cos(ang)
    ws = np.sin(ang)
    wc[2049:] = 0.0
    ws[2049:] = 0.0
    cv = np.full(FP, 2.0)
    cv[0] = 1.0
    cv[2048] = 1.0
    cv[2049:] = 0.0
    cv /= float(L) * float(D)
    cv3 = cv.reshape(NWB, 1, WBLK).astype(np.float32)
    wch, wcl = _split_np(wc)
    wsh, wsl = _split_np(ws)
    return wch, wcl, wsh, wsl, cv3


def _dot(a, b):
    return lax.dot_general(a, b, (((1,), (0,)), ((), ())),
                           preferred_element_type=jnp.float32)


def _dot3(ah, al, bh, bl):
    return _dot(ah, bh) + _dot(ah, bl) + _dot(al, bh)


def _split_jx(x):
    hi = x.astype(jnp.bfloat16)
    lo = (x - hi.astype(jnp.float32)).astype(jnp.bfloat16)
    return hi, lo


DBLK = 512
NDB = D // DBLK


def _spectrum_body(qh_ref, ql_ref, kh_ref, kl_ref,
                   wch_ref, wcl_ref, wsh_ref, wsl_ref, sr_ref, si_ref,
                   asr_ref, asi_ref):
    d2 = pl.program_id(1)
    w = pl.program_id(2)
    qh, ql = qh_ref[0], ql_ref[0]
    kh, kl = kh_ref[0], kl_ref[0]
    wch, wcl = wch_ref[...], wcl_ref[...]
    wsh, wsl = wsh_ref[...], wsl_ref[...]
    qr = _dot3(wch, wcl, qh, ql)
    qs = _dot3(wsh, wsl, qh, ql)
    kr = _dot3(wch, wcl, kh, kl)
    ks = _dot3(wsh, wsl, kh, kl)
    sr = jnp.sum(qr * kr + qs * ks, axis=1, keepdims=True)
    si = jnp.sum(qr * ks - qs * kr, axis=1, keepdims=True)
    rows = pl.ds(w * WBLK, WBLK)

    @pl.when(d2 == 0)
    def _():
        asr_ref[rows, :] = sr
        asi_ref[rows, :] = si

    @pl.when(d2 > 0)
    def _():
        asr_ref[rows, :] = asr_ref[rows, :] + sr
        asi_ref[rows, :] = asi_ref[rows, :] + si

    @pl.when(d2 == NDB - 1)
    def _():
        sr_ref[0] = asr_ref[rows, :]
        si_ref[0] = asi_ref[rows, :]


def _cross_spectrum(qh, ql, kh, kl, wch, wcl, wsh, wsl):
    grid = (B, NDB, NWB)
    dspec = pl.BlockSpec((1, L, DBLK), lambda b, d2, w: (b, 0, d2))
    wspec = pl.BlockSpec((WBLK, L), lambda b, d2, w: (w, 0))
    return pl.pallas_call(
        _spectrum_body,
        grid=grid,
        in_specs=[dspec, dspec, dspec, dspec, wspec, wspec, wspec, wspec],
        out_specs=[
            pl.BlockSpec((1, WBLK, 1), lambda b, d2, w: (b, w, 0)),
            pl.BlockSpec((1, WBLK, 1), lambda b, d2, w: (b, w, 0)),
        ],
        out_shape=[
            jax.ShapeDtypeStruct((B, FP, 1), jnp.float32),
            jax.ShapeDtypeStruct((B, FP, 1), jnp.float32),
        ],
        scratch_shapes=[
            pltpu.VMEM((FP, 1), jnp.float32),
            pltpu.VMEM((FP, 1), jnp.float32),
        ],
    )(qh, ql, kh, kl, wch, wcl, wsh, wsl)


def _select_body(sr_ref, si_ref, cv_ref, wch_ref, wcl_ref, wsh_ref, wsl_ref,
                 ti_ref, wt_ref, acc_ref):
    w = pl.program_id(0)
    srh, srl = _split_jx(sr_ref[0] * cv_ref[0])
    sih, sil = _split_jx(si_ref[0] * cv_ref[0])
    contrib = (_dot3(srh, srl, wch_ref[...], wcl_ref[...])
               - _dot3(sih, sil, wsh_ref[...], wsl_ref[...]))

    @pl.when(w == 0)
    def _():
        acc_ref[...] = contrib

    @pl.when(w > 0)
    def _():
        acc_ref[...] = acc_ref[...] + contrib

    @pl.when(w == NWB - 1)
    def _():
        scores = acc_ref[...]
        iota = lax.broadcasted_iota(jnp.int32, (B, L), 1)
        cur = scores
        vals, idxs = [], []
        for _ in range(KTOP):
            m = jnp.max(cur, axis=1, keepdims=True)
            hit = cur == m
            idx = jnp.min(jnp.where(hit, iota, L), axis=1, keepdims=True)
            vals.append(m)
            idxs.append(idx)
            cur = jnp.where(iota == idx, -1e30, cur)
        pad_v = [jnp.full((B, 1), -1e30, jnp.float32)] * (8 - KTOP)
        pad_i = [jnp.zeros((B, 1), jnp.int32)] * (8 - KTOP)
        v = jnp.concatenate(vals + pad_v, axis=1)
        ix = jnp.concatenate(idxs + pad_i, axis=1)
        mx = jnp.max(v, axis=1, keepdims=True)
        e = jnp.exp(v - mx)
        wt = e / jnp.sum(e, axis=1, keepdims=True)
        s_b = lax.broadcast_in_dim(ix, (B, 8, L), (0, 1))
        l_i = lax.broadcasted_iota(jnp.int32, (B, 8, L), 2)
        b_i = lax.broadcasted_iota(jnp.int32, (B, 8, L), 0)
        ti_ref[...] = b_i * L + lax.rem(l_i - s_b + L, L)
        wt_ref[...] = lax.broadcast_in_dim(wt, (B, 8, 16), (0, 1))


def _select(sr, si, cv, wch, wcl, wsh, wsl):
    grid = (NWB,)
    wspec = pl.BlockSpec((WBLK, L), lambda w: (w, 0))
    return pl.pallas_call(
        _select_body,
        grid=grid,
        in_specs=[
            pl.BlockSpec((1, B, WBLK), lambda w: (w, 0, 0)),
            pl.BlockSpec((1, B, WBLK), lambda w: (w, 0, 0)),
            pl.BlockSpec((1, 1, WBLK), lambda w: (w, 0, 0)),
            wspec, wspec, wspec, wspec,
        ],
        out_specs=[
            pl.BlockSpec((B, 8, L), lambda w: (0, 0, 0)),
            pl.BlockSpec((B, 8, 16), lambda w: (0, 0, 0)),
        ],
        out_shape=[
            jax.ShapeDtypeStruct((B, 8, L), jnp.int32),
            jax.ShapeDtypeStruct((B, 8, 16), jnp.float32),
        ],
        scratch_shapes=[pltpu.VMEM((B, L), jnp.float32)],
    )(sr, si, cv, wch, wcl, wsh, wsl)


NWORK = 32
RPW = (B * L) // NWORK
CH = 16
NCH = RPW // CH


def _roll_body(vals_hbm, im_hbm, w_hbm, out_hbm,
               w0_v, w1_v, w2_v, w3_v, w4_v,
               idx0, idx1, buf0, buf1, acc, sem0, sem1):
    cid = lax.axis_index("c")
    sid = lax.axis_index("s")
    wid = sid * 2 + cid
    b = wid // (NWORK // B)
    blk = wid % (NWORK // B)
    w_slots = [w0_v, w1_v, w2_v, w3_v, w4_v]
    for i in range(KTOP):
        pltpu.sync_copy(w_hbm.at[b * 8 + i], w_slots[i])
    wvs = [w_slots[i][...] for i in range(KTOP)]
    base = blk * RPW
    row0 = b * L + base
    imbase = b * 8 * L + base
    idxs_ = [idx0, idx1]
    bufs_ = [buf0, buf1]
    sems_ = [sem0, sem1]

    def start(ci, i, p):
        pltpu.sync_copy(im_hbm.at[pl.ds(imbase + i * L + ci * CH, CH)],
                        idxs_[p])
        pltpu.async_copy(vals_hbm.at[idxs_[p]], bufs_[p], sems_[p])

    def wait_for(p):
        pltpu.make_async_copy(vals_hbm.at[pl.ds(0, CH)], bufs_[p],
                              sems_[p]).wait()

    def accumulate(bufref, wv, i):
        def rb(r, c):
            for g in range(D // 16):
                sl = pl.ds(g * 16, 16)
                if i == 0:
                    bufref_val = bufref[r, sl] * wv
                    acc[r, sl] = bufref_val
                else:
                    acc[r, sl] = acc[r, sl] + bufref[r, sl] * wv
            return c
        lax.fori_loop(0, CH, rb, 0)

    start(0, 0, 0)

    def pair_body(cp, carry):
        for j in range(2 * KTOP):
            ci = 2 * cp + (1 if j >= KTOP else 0)
            i = j % KTOP
            p = j % 2
            if j < 2 * KTOP - 1:
                nj = j + 1
                start(2 * cp + (1 if nj >= KTOP else 0), nj % KTOP,
                      nj % 2)
            else:
                @pl.when(cp < NCH // 2 - 1)
                def _():
                    start(2 * cp + 2, 0, 0)
            wait_for(p)
            accumulate(bufs_[p], wvs[i], i)
            if i == KTOP - 1:
                pltpu.sync_copy(acc, out_hbm.at[pl.ds(row0 + ci * CH, CH)])
        return carry

    lax.fori_loop(0, NCH // 2, pair_body, 0)


def _sc_roll(values2d, idxmap, wts):
    mesh = plsc.VectorSubcoreMesh(core_axis_name="c", subcore_axis_name="s")
    fn = functools.partial(
        pl.kernel,
        mesh=mesh,
        out_type=jax.ShapeDtypeStruct((B * L, D), jnp.float32),
        scratch_types=[
            pltpu.VMEM((16,), jnp.float32),
            pltpu.VMEM((16,), jnp.float32),
            pltpu.VMEM((16,), jnp.float32),
            pltpu.VMEM((16,), jnp.float32),
            pltpu.VMEM((16,), jnp.float32),
            pltpu.VMEM((CH,), jnp.int32),
            pltpu.VMEM((CH,), jnp.int32),
            pltpu.VMEM((CH, D), jnp.float32),
            pltpu.VMEM((CH, D), jnp.float32),
            pltpu.VMEM((CH, D), jnp.float32),
            pltpu.SemaphoreType.DMA,
            pltpu.SemaphoreType.DMA,
        ],
    )(_roll_body)
    return fn(values2d, idxmap, wts)


def kernel(queries, keys, values):
    wch_np, wcl_np, wsh_np, wsl_np, cv_np = _dft_consts()
    wch, wcl = jnp.asarray(wch_np), jnp.asarray(wcl_np)
    wsh, wsl = jnp.asarray(wsh_np), jnp.asarray(wsl_np)
    cv = jnp.asarray(cv_np)
    qh, ql = _split_jx(queries)
    kh, kl = _split_jx(keys)
    sr, si = _cross_spectrum(qh, ql, kh, kl, wch, wcl, wsh, wsl)
    sr3 = jnp.transpose(sr.reshape(B, NWB, WBLK), (1, 0, 2))
    si3 = jnp.transpose(si.reshape(B, NWB, WBLK), (1, 0, 2))
    idxmap, wts = _select(sr3, si3, cv, wch, wcl, wsh, wsl)
    out2d = _sc_roll(values.reshape(B * L, D),
                     idxmap.reshape(B * 8 * L), wts.reshape(B * 8, 16))
    return out2d.reshape(B, L, D)


# speedup vs baseline: 2.5480x; 1.0126x over previous
import functools

import numpy as np
import jax
import jax.numpy as jnp
from jax import lax
from jax.experimental import pallas as pl
from jax.experimental.pallas import tpu as pltpu
from jax.experimental.pallas import tpu_sc as plsc

B = 4
L = 4096
D = 1024
KTOP = 5
FP = 2112
WBLK = 176
NWB = FP // WBLK

def _split_np(x):
    hi = x.astype(np.float32).astype(jnp.bfloat16)
    lo = (x.astype(np.float32) - hi.astype(np.float32)).astype(jnp.bfloat16)
    return hi, lo


@functools.lru_cache(maxsize=None)
def _dft_consts():
    f = np.arange(FP, dtype=np.float64)[:, None]
    t = np.arange(L, dtype=np.float64)[None, :]
    ang = 2.0 * np.pi * f * t / L
    wc = np.cos(ang)
    ws = np.sin(ang)
    wc[2049:] = 0.0
    ws[2049:] = 0.0
    cv = np.full(FP, 2.0)
    cv[0] = 1.0
    cv[2048] = 1.0
    cv[2049:] = 0.0
    cv /= float(L) * float(D)
    cv3 = cv.reshape(NWB, 1, WBLK).astype(np.float32)
    wch, wcl = _split_np(wc)
    wsh, wsl = _split_np(ws)
    return wch, wcl, wsh, wsl, cv3


def _dot(a, b):
    return lax.dot_general(a, b, (((1,), (0,)), ((), ())),
                           preferred_element_type=jnp.float32)


def _dot3(ah, al, bh, bl):
    return _dot(ah, bh) + _dot(ah, bl) + _dot(al, bh)


def _split_jx(x):
    hi = x.astype(jnp.bfloat16)
    lo = (x - hi.astype(jnp.float32)).astype(jnp.bfloat16)
    return hi, lo


DBLK = 512
NDB = D // DBLK


def _spectrum_body(qh_ref, ql_ref, kh_ref, kl_ref,
                   wch_ref, wcl_ref, wsh_ref, wsl_ref, sr_ref, si_ref,
                   asr_ref, asi_ref):
    d2 = pl.program_id(1)
    w = pl.program_id(2)
    qh, ql = qh_ref[0], ql_ref[0]
    kh, kl = kh_ref[0], kl_ref[0]
    wch, wcl = wch_ref[...], wcl_ref[...]
    wsh, wsl = wsh_ref[...], wsl_ref[...]
    qr = _dot3(wch, wcl, qh, ql)
    qs = _dot3(wsh, wsl, qh, ql)
    kr = _dot3(wch, wcl, kh, kl)
    ks = _dot3(wsh, wsl, kh, kl)
    sr = jnp.sum(qr * kr + qs * ks, axis=1, keepdims=True)
    si = jnp.sum(qr * ks - qs * kr, axis=1, keepdims=True)
    rows = pl.ds(w * WBLK, WBLK)

    @pl.when(d2 == 0)
    def _():
        asr_ref[rows, :] = sr
        asi_ref[rows, :] = si

    @pl.when(d2 > 0)
    def _():
        asr_ref[rows, :] = asr_ref[rows, :] + sr
        asi_ref[rows, :] = asi_ref[rows, :] + si

    @pl.when(d2 == NDB - 1)
    def _():
        sr_ref[0] = asr_ref[rows, :]
        si_ref[0] = asi_ref[rows, :]


def _cross_spectrum(qh, ql, kh, kl, wch, wcl, wsh, wsl):
    grid = (B, NDB, NWB)
    dspec = pl.BlockSpec((1, L, DBLK), lambda b, d2, w: (b, 0, d2))
    wspec = pl.BlockSpec((WBLK, L), lambda b, d2, w: (w, 0))
    return pl.pallas_call(
        _spectrum_body,
        grid=grid,
        in_specs=[dspec, dspec, dspec, dspec, wspec, wspec, wspec, wspec],
        out_specs=[
            pl.BlockSpec((1, WBLK, 1), lambda b, d2, w: (b, w, 0)),
            pl.BlockSpec((1, WBLK, 1), lambda b, d2, w: (b, w, 0)),
        ],
        out_shape=[
            jax.ShapeDtypeStruct((B, FP, 1), jnp.float32),
            jax.ShapeDtypeStruct((B, FP, 1), jnp.float32),
        ],
        scratch_shapes=[
            pltpu.VMEM((FP, 1), jnp.float32),
            pltpu.VMEM((FP, 1), jnp.float32),
        ],
    )(qh, ql, kh, kl, wch, wcl, wsh, wsl)


def _select_body(sr_ref, si_ref, cv_ref, wch_ref, wcl_ref, wsh_ref, wsl_ref,
                 ti_ref, wt_ref, acc_ref):
    w = pl.program_id(0)
    srh, srl = _split_jx(sr_ref[0] * cv_ref[0])
    sih, sil = _split_jx(si_ref[0] * cv_ref[0])
    contrib = (_dot3(srh, srl, wch_ref[...], wcl_ref[...])
               - _dot3(sih, sil, wsh_ref[...], wsl_ref[...]))

    @pl.when(w == 0)
    def _():
        acc_ref[...] = contrib

    @pl.when(w > 0)
    def _():
        acc_ref[...] = acc_ref[...] + contrib

    @pl.when(w == NWB - 1)
    def _():
        scores = acc_ref[...]
        iota = lax.broadcasted_iota(jnp.int32, (B, L), 1)
        cur = scores
        vals, idxs = [], []
        for _ in range(KTOP):
            m = jnp.max(cur, axis=1, keepdims=True)
            hit = cur == m
            idx = jnp.min(jnp.where(hit, iota, L), axis=1, keepdims=True)
            vals.append(m)
            idxs.append(idx)
            cur = jnp.where(iota == idx, -1e30, cur)
        pad_v = [jnp.full((B, 1), -1e30, jnp.float32)] * (8 - KTOP)
        pad_i = [jnp.zeros((B, 1), jnp.int32)] * (8 - KTOP)
        v = jnp.concatenate(vals + pad_v, axis=1)
        ix = jnp.concatenate(idxs + pad_i, axis=1)
        mx = jnp.max(v, axis=1, keepdims=True)
        e = jnp.exp(v - mx)
        wt = e / jnp.sum(e, axis=1, keepdims=True)
        sh = (B, L // 8, 8, 8)
        s_b = lax.broadcast_in_dim(ix, sh, (0, 2))
        c_i = lax.broadcasted_iota(jnp.int32, sh, 1)
        r_i = lax.broadcasted_iota(jnp.int32, sh, 3)
        b_i = lax.broadcasted_iota(jnp.int32, sh, 0)
        ti_ref[...] = b_i * L + lax.rem(c_i * 8 + r_i - s_b + L, L)
        wt_ref[...] = lax.broadcast_in_dim(wt, (B, 8, 16), (0, 1))


def _select(sr, si, cv, wch, wcl, wsh, wsl):
    grid = (NWB,)
    wspec = pl.BlockSpec((WBLK, L), lambda w: (w, 0))
    return pl.pallas_call(
        _select_body,
        grid=grid,
        in_specs=[
            pl.BlockSpec((1, B, WBLK), lambda w: (w, 0, 0)),
            pl.BlockSpec((1, B, WBLK), lambda w: (w, 0, 0)),
            pl.BlockSpec((1, 1, WBLK), lambda w: (w, 0, 0)),
            wspec, wspec, wspec, wspec,
        ],
        out_specs=[
            pl.BlockSpec((B, L // 8, 8, 8), lambda w: (0, 0, 0, 0)),
            pl.BlockSpec((B, 8, 16), lambda w: (0, 0, 0)),
        ],
        out_shape=[
            jax.ShapeDtypeStruct((B, L // 8, 8, 8), jnp.int32),
            jax.ShapeDtypeStruct((B, 8, 16), jnp.float32),
        ],
        scratch_shapes=[pltpu.VMEM((B, L), jnp.float32)],
    )(sr, si, cv, wch, wcl, wsh, wsl)


NWORK = 32
RPW = (B * L) // NWORK
CH = 8
NCH = RPW // CH


def _roll_body(vals_hbm, im_hbm, w_hbm, out_hbm,
               w0_v, w1_v, w2_v, w3_v, w4_v,
               idx0, idx1,
               a0, a1, a2, a3, a4, b0, b1, b2, b3, b4,
               acc, sem0, sem1):
    cid = lax.axis_index("c")
    sid = lax.axis_index("s")
    wid = sid * 2 + cid
    b = wid // (NWORK // B)
    blk = wid % (NWORK // B)
    w_slots = [w0_v, w1_v, w2_v, w3_v, w4_v]
    for i in range(KTOP):
        pltpu.sync_copy(w_hbm.at[b * 8 + i], w_slots[i])
    wvs = [w_slots[i][...] for i in range(KTOP)]
    base = blk * RPW
    row0 = b * L + base
    c0 = (b * L + base) // CH
    idxs_ = [idx0, idx1]
    bufs_ = [[a0, a1, a2, a3, a4], [b0, b1, b2, b3, b4]]
    sems_ = [sem0, sem1]

    def start(ci, p):
        pltpu.sync_copy(im_hbm.at[pl.ds((c0 + ci) * 64, 64)], idxs_[p])
        for i in range(KTOP):
            pltpu.async_copy(vals_hbm.at[idxs_[p].at[pl.ds(i * CH, CH)]],
                             bufs_[p][i], sems_[p])

    def wait_for(p):
        for i in range(KTOP):
            pltpu.make_async_copy(vals_hbm.at[pl.ds(0, CH)], bufs_[p][i],
                                  sems_[p]).wait()

    def step(ci, p, prefetch):
        if prefetch is not None:
            ni, np_ = prefetch
            start(ni, np_)
        else:
            pass
        wait_for(p)
        bs = bufs_[p]

        def rb(r, c):
            for g in range(D // 16):
                sl = pl.ds(g * 16, 16)
                x = bs[0][r, sl] * wvs[0]
                for i in range(1, KTOP):
                    x = x + bs[i][r, sl] * wvs[i]
                acc[r, sl] = x
            return c
        lax.fori_loop(0, CH, rb, 0)
        pltpu.sync_copy(acc, out_hbm.at[pl.ds(row0 + ci * CH, CH)])

    start(0, 0)

    def pair_body(cp, carry):
        ci = 2 * cp
        step(ci, 0, (ci + 1, 1))
        @pl.when(cp < NCH // 2 - 1)
        def _():
            start(ci + 2, 0)
        step(ci + 1, 1, None)
        return carry

    lax.fori_loop(0, NCH // 2, pair_body, 0)


def _sc_roll(values2d, idxmap, wts):
    mesh = plsc.VectorSubcoreMesh(core_axis_name="c", subcore_axis_name="s")
    fn = functools.partial(
        pl.kernel,
        mesh=mesh,
        out_type=jax.ShapeDtypeStruct((B * L, D), jnp.float32),
        scratch_types=(
            [pltpu.VMEM((16,), jnp.float32)] * 5
            + [pltpu.VMEM((64,), jnp.int32)] * 2
            + [pltpu.VMEM((CH, D), jnp.float32)] * 11
            + [pltpu.SemaphoreType.DMA] * 2
        ),
    )(_roll_body)
    return fn(values2d, idxmap, wts)


def kernel(queries, keys, values):
    wch_np, wcl_np, wsh_np, wsl_np, cv_np = _dft_consts()
    wch, wcl = jnp.asarray(wch_np), jnp.asarray(wcl_np)
    wsh, wsl = jnp.asarray(wsh_np), jnp.asarray(wsl_np)
    cv = jnp.asarray(cv_np)
    qh, ql = _split_jx(queries)
    kh, kl = _split_jx(keys)
    sr, si = _cross_spectrum(qh, ql, kh, kl, wch, wcl, wsh, wsl)
    sr3 = jnp.transpose(sr.reshape(B, NWB, WBLK), (1, 0, 2))
    si3 = jnp.transpose(si.reshape(B, NWB, WBLK), (1, 0, 2))
    idxmap, wts = _select(sr3, si3, cv, wch, wcl, wsh, wsl)
    out2d = _sc_roll(values.reshape(B * L, D),
                     idxmap.reshape(B * L * 8), wts.reshape(B * 8, 16))
    return out2d.reshape(B, L, D)
